# baseline (speedup 1.0000x reference)
import functools

import jax
import jax.numpy as jnp
from jax import lax
from jax.experimental import pallas as pl
from jax.experimental.pallas import tpu as pltpu
from jax.experimental.pallas import tpu_sc as plsc

_ALPHA = 3.0
_MIN_DIFF = 0.1
_N = 4096

_NC = 2
_NS = 16
_L = 16
_NW = _NC * _NS
_ROWS_PER_W = _N // _NW

_LOG1P_COEF = (
    -6.0066050e-03, 3.4264602e-02, -9.2290416e-02, 1.6499813e-01,
    -2.3943338e-01, 3.3144665e-01, -4.9982551e-01, 9.9999362e-01,
    3.9109054e-08,
)


def _log1p_poly(u):
    r = jnp.full_like(u, _LOG1P_COEF[0])
    for c in _LOG1P_COEF[1:]:
        r = r * u + jnp.float32(c)
    return r


_GATHER_DNUMS = lax.GatherDimensionNumbers(
    offset_dims=(), collapsed_slice_dims=(0,), start_index_map=(0,))


def _bcast_lane(v, r):
    idx = jnp.full((_L, 1), r, jnp.int32)
    return lax.gather(v, idx, _GATHER_DNUMS, (1,),
                      mode=lax.GatherScatterMode.PROMISE_IN_BOUNDS)


_RB = 8


def _sc_body(pred_hbm, target_hbm, loss_hbm, cnt_hbm, reg_hbm,
             p_v, t_v, out_s):
    wid = lax.axis_index("s") * _NC + lax.axis_index("c")
    pltpu.sync_copy(pred_hbm, p_v.at[pl.ds(0, _N)])
    pltpu.sync_copy(target_hbm, t_v.at[pl.ds(0, _N)])
    base = wid * _ROWS_PER_W
    zero = jnp.zeros((_L,), jnp.float32)

    def blk_body(b, carry):
        row0 = base + b * _RB
        vrow_p = p_v[pl.ds(row0, _L)]
        vrow_t = t_v[pl.ds(row0, _L)]
        p_b = [_bcast_lane(vrow_p, r) for r in range(_RB)]
        t_b = [_bcast_lane(vrow_t, r) for r in range(_RB)]

        def col_body(cidx, carry2):
            laccs, caccs = carry2
            j0 = cidx * _L
            vp = p_v[pl.ds(j0, _L)]
            vt = t_v[pl.ds(j0, _L)]
            new_l, new_c = [], []
            for r in range(_RB):
                d = t_b[r] - vt
                dp = p_b[r] - vp
                w = d * dp
                e = jnp.exp(-jnp.abs(dp))
                c = jnp.abs(d)
                term = jnp.maximum(-w, 0.0) + c * _log1p_poly(e)
                maskf = jnp.where(c > _MIN_DIFF, 1.0, 0.0)
                new_l.append(laccs[r] + maskf * term)
                new_c.append(caccs[r] + maskf)
            return tuple(new_l), tuple(new_c)

        return lax.fori_loop(0, _N // _L, col_body, carry)

    init = (tuple(zero for _ in range(_RB)), tuple(zero for _ in range(_RB)))
    laccs, caccs = lax.fori_loop(0, _ROWS_PER_W // _RB, blk_body, init)
    lacc = laccs[0]
    cacc = caccs[0]
    for r in range(1, _RB):
        lacc = lacc + laccs[r]
        cacc = cacc + caccs[r]

    def reg_body(k, racc):
        j0 = base + k * _L
        e = p_v[pl.ds(j0, _L)] - t_v[pl.ds(j0, _L)]
        return racc + e * e

    racc = lax.fori_loop(0, _ROWS_PER_W // _L, reg_body, zero)

    out_s[0, :] = lacc
    out_s[1, :] = cacc
    out_s[2, :] = racc
    pltpu.sync_copy(out_s.at[0], loss_hbm.at[wid])
    pltpu.sync_copy(out_s.at[1], cnt_hbm.at[wid])
    pltpu.sync_copy(out_s.at[2], reg_hbm.at[wid])


_sc_program = functools.partial(
    pl.kernel,
    out_type=[
        jax.ShapeDtypeStruct((_NW, _L), jnp.float32),
        jax.ShapeDtypeStruct((_NW, _L), jnp.float32),
        jax.ShapeDtypeStruct((_NW, _L), jnp.float32),
    ],
    mesh=plsc.VectorSubcoreMesh(core_axis_name="c", subcore_axis_name="s"),
    scratch_types=[
        pltpu.VMEM((_N + _L,), jnp.float32),
        pltpu.VMEM((_N + _L,), jnp.float32),
        pltpu.VMEM((3, _L), jnp.float32),
    ],
)(_sc_body)


@jax.jit
def kernel(pred, target):
    p = pred.reshape(_N)
    t = target.reshape(_N)
    loss_part, cnt_part, reg_part = _sc_program(p, t)
    loss_sum = jnp.sum(loss_part)
    cnt = jnp.sum(cnt_part)
    reg = jnp.sum(reg_part) / _N
    pair_mean = loss_sum / jnp.maximum(cnt, 1.0)
    return jnp.where(cnt > 0, reg + _ALPHA * pair_mean, reg)


# speedup vs baseline: 2.7998x; 2.7998x over previous
import functools

import jax
import jax.numpy as jnp
from jax.experimental import pallas as pl

_ALPHA = 3.0
_MIN_DIFF = 0.1
_N = 4096

_BR = 256
_BC = 1024


def _mserank_tile(p_col_ref, t_col_ref, p_row_ref, t_row_ref,
                  loss_ref, cnt_ref, reg_ref):
    ri = pl.program_id(0)
    ci = pl.program_id(1)

    @pl.when(jnp.logical_and(ri == 0, ci == 0))
    def _init():
        loss_ref[...] = jnp.zeros((1, 1), jnp.float32)
        cnt_ref[...] = jnp.zeros((1, 1), jnp.float32)
        reg_ref[...] = jnp.zeros((1, 1), jnp.float32)

    p_i = p_col_ref[...]
    t_i = t_col_ref[...]
    p_j = p_row_ref[...]
    t_j = t_row_ref[...]

    d = t_i - t_j
    dp = p_i - p_j
    c = jnp.abs(d)
    term = jnp.maximum(-d * dp, 0.0) + c * jnp.log1p(jnp.exp(-jnp.abs(dp)))
    maskf = jnp.where(c > _MIN_DIFF, 1.0, 0.0)
    loss_ref[...] += jnp.sum(maskf * term, keepdims=True)
    cnt_ref[...] += jnp.sum(maskf, keepdims=True)

    @pl.when(ci == 0)
    def _reg():
        e = p_i - t_i
        reg_ref[...] += jnp.sum(e * e, keepdims=True)


@jax.jit
def kernel(pred, target):
    p = pred.reshape(_N, 1)
    t = target.reshape(_N, 1)
    p_row = pred.reshape(1, _N)
    t_row = target.reshape(1, _N)

    grid = (_N // _BR, _N // _BC)
    loss_sum, cnt, reg_sum = pl.pallas_call(
        _mserank_tile,
        grid=grid,
        in_specs=[
            pl.BlockSpec((_BR, 1), lambda r, c: (r, 0)),
            pl.BlockSpec((_BR, 1), lambda r, c: (r, 0)),
            pl.BlockSpec((1, _BC), lambda r, c: (0, c)),
            pl.BlockSpec((1, _BC), lambda r, c: (0, c)),
        ],
        out_specs=[
            pl.BlockSpec((1, 1), lambda r, c: (0, 0)),
            pl.BlockSpec((1, 1), lambda r, c: (0, 0)),
            pl.BlockSpec((1, 1), lambda r, c: (0, 0)),
        ],
        out_shape=[
            jax.ShapeDtypeStruct((1, 1), jnp.float32),
            jax.ShapeDtypeStruct((1, 1), jnp.float32),
            jax.ShapeDtypeStruct((1, 1), jnp.float32),
        ],
    )(p, t, p_row, t_row)

    loss_sum = loss_sum[0, 0]
    cnt = cnt[0, 0]
    reg = reg_sum[0, 0] / _N
    pair_mean = loss_sum / jnp.maximum(cnt, 1.0)
    return jnp.where(cnt > 0, reg + _ALPHA * pair_mean, reg)
